# Initial kernel scaffold; baseline (speedup 1.0000x reference)
import jax
import jax.numpy as jnp
from jax.experimental import pallas as pl


def kernel(sample, entity_embedding, relation_embedding):
    raise NotImplementedError("write your pallas kernel here")


# speedup vs baseline: 1.0544x; 1.0544x over previous
import functools

import jax
import jax.numpy as jnp
from jax import lax
from jax.experimental import pallas as pl
from jax.experimental.pallas import tpu as pltpu
from jax.experimental.pallas import tpu_sc as plsc

GAMMA = 12.0
BATCH = 4096
DIM = 128
LANES = 16
NUM_CORES = 2
NUM_SUBCORES = 16
NW = NUM_CORES * NUM_SUBCORES
BPW = BATCH // NW
CHUNKS = DIM // LANES


def _transe_body(entity_hbm, relation_hbm, hidx_hbm, ridx_hbm, tidx_hbm,
                 out_hbm,
                 hidx_v, ridx_v, tidx_v, h_rows, r_rows, t_rows,
                 out_v, sem):
    wid = lax.axis_index("s") * NUM_CORES + lax.axis_index("c")
    base = wid * BPW

    pltpu.sync_copy(hidx_hbm.at[pl.ds(base, BPW)], hidx_v)
    pltpu.sync_copy(ridx_hbm.at[pl.ds(base, BPW)], ridx_v)
    pltpu.sync_copy(tidx_hbm.at[pl.ds(base, BPW)], tidx_v)

    ch = pltpu.async_copy(entity_hbm.at[hidx_v], h_rows, sem)
    cr = pltpu.async_copy(relation_hbm.at[ridx_v], r_rows, sem)
    ct = pltpu.async_copy(entity_hbm.at[tidx_v], t_rows, sem)
    ch.wait()
    cr.wait()
    ct.wait()

    lane = lax.iota(jnp.int32, LANES)

    def group_body(g, carry):
        score = jnp.zeros((LANES,), jnp.float32)
        for l in range(LANES):
            i = g * LANES + l
            acc = jnp.zeros((LANES,), jnp.float32)
            for c in range(CHUNKS):
                h = h_rows[i, pl.ds(c * LANES, LANES)]
                r = r_rows[i, pl.ds(c * LANES, LANES)]
                t = t_rows[i, pl.ds(c * LANES, LANES)]
                acc = acc + jnp.abs(h + r - t)
            score = jnp.where(lane == l, jnp.sum(acc), score)
        out_v[pl.ds(g * LANES, LANES)] = GAMMA - score
        return carry

    lax.fori_loop(0, BPW // LANES, group_body, 0)

    pltpu.sync_copy(out_v, out_hbm.at[pl.ds(base, BPW)])


_transe_sc = functools.partial(
    pl.kernel,
    mesh=plsc.VectorSubcoreMesh(core_axis_name="c", subcore_axis_name="s"),
    out_type=jax.ShapeDtypeStruct((BATCH,), jnp.float32),
    compiler_params=pltpu.CompilerParams(needs_layout_passes=False),
    scratch_types=[
        pltpu.VMEM((BPW,), jnp.int32),
        pltpu.VMEM((BPW,), jnp.int32),
        pltpu.VMEM((BPW,), jnp.int32),
        pltpu.VMEM((BPW, DIM), jnp.float32),
        pltpu.VMEM((BPW, DIM), jnp.float32),
        pltpu.VMEM((BPW, DIM), jnp.float32),
        pltpu.VMEM((BPW,), jnp.float32),
        pltpu.SemaphoreType.DMA,
    ],
)(_transe_body)


@jax.jit
def kernel(sample, entity_embedding, relation_embedding):
    hidx = sample[:, 0]
    ridx = sample[:, 1]
    tidx = sample[:, 2]
    score = _transe_sc(entity_embedding, relation_embedding, hidx, ridx, tidx)
    return score.reshape(BATCH, 1)


# speedup vs baseline: 1.3397x; 1.2707x over previous
import functools

import jax
import jax.numpy as jnp
from jax import lax
from jax.experimental import pallas as pl
from jax.experimental.pallas import tpu as pltpu
from jax.experimental.pallas import tpu_sc as plsc

GAMMA = 12.0
BATCH = 4096
DIM = 128
LANES = 16
NUM_CORES = 2
NUM_SUBCORES = 16
NW = NUM_CORES * NUM_SUBCORES
BPW = BATCH // NW
CHUNKS = DIM // LANES
STRIDE = LANES + 1


def _transe_body(entity_hbm, relation_hbm, hidx_hbm, ridx_hbm, tidx_hbm,
                 out_hbm,
                 hidx_v, ridx_v, tidx_v, h_rows, r_rows, t_rows,
                 partials, out_v, sem):
    wid = lax.axis_index("s") * NUM_CORES + lax.axis_index("c")
    base = wid * BPW

    ci0 = pltpu.async_copy(hidx_hbm.at[pl.ds(base, BPW)], hidx_v, sem)
    ci1 = pltpu.async_copy(ridx_hbm.at[pl.ds(base, BPW)], ridx_v, sem)
    ci2 = pltpu.async_copy(tidx_hbm.at[pl.ds(base, BPW)], tidx_v, sem)
    ci0.wait()
    ci1.wait()
    ci2.wait()

    ch = pltpu.async_copy(entity_hbm.at[hidx_v], h_rows, sem)
    cr = pltpu.async_copy(relation_hbm.at[ridx_v], r_rows, sem)
    ct = pltpu.async_copy(entity_hbm.at[tidx_v], t_rows, sem)
    ch.wait()
    cr.wait()
    ct.wait()

    def row_body(i, carry):
        acc0 = jnp.zeros((LANES,), jnp.float32)
        acc1 = jnp.zeros((LANES,), jnp.float32)
        for c in range(0, CHUNKS, 2):
            h = h_rows[i, pl.ds(c * LANES, LANES)]
            r = r_rows[i, pl.ds(c * LANES, LANES)]
            t = t_rows[i, pl.ds(c * LANES, LANES)]
            acc0 = acc0 + jnp.abs(h + r - t)
            h = h_rows[i, pl.ds((c + 1) * LANES, LANES)]
            r = r_rows[i, pl.ds((c + 1) * LANES, LANES)]
            t = t_rows[i, pl.ds((c + 1) * LANES, LANES)]
            acc1 = acc1 + jnp.abs(h + r - t)
        partials[pl.ds(i * STRIDE, LANES)] = acc0 + acc1
        return carry

    lax.fori_loop(0, BPW, row_body, 0, unroll=2)

    lane = lax.iota(jnp.int32, LANES)
    for g in range(BPW // LANES):
        rows = (lane + g * LANES) * STRIDE
        tot = jnp.zeros((LANES,), jnp.float32)
        for c in range(LANES):
            tot = tot + plsc.load_gather(partials, [rows + c])
        out_v[pl.ds(g * LANES, LANES)] = GAMMA - tot

    pltpu.sync_copy(out_v, out_hbm.at[pl.ds(base, BPW)])


_transe_sc = functools.partial(
    pl.kernel,
    mesh=plsc.VectorSubcoreMesh(core_axis_name="c", subcore_axis_name="s"),
    out_type=jax.ShapeDtypeStruct((BATCH,), jnp.float32),
    compiler_params=pltpu.CompilerParams(needs_layout_passes=False),
    scratch_types=[
        pltpu.VMEM((BPW,), jnp.int32),
        pltpu.VMEM((BPW,), jnp.int32),
        pltpu.VMEM((BPW,), jnp.int32),
        pltpu.VMEM((BPW, DIM), jnp.float32),
        pltpu.VMEM((BPW, DIM), jnp.float32),
        pltpu.VMEM((BPW, DIM), jnp.float32),
        pltpu.VMEM((BPW * STRIDE,), jnp.float32),
        pltpu.VMEM((BPW,), jnp.float32),
        pltpu.SemaphoreType.DMA,
    ],
)(_transe_body)


@jax.jit
def kernel(sample, entity_embedding, relation_embedding):
    hidx = sample[:, 0]
    ridx = sample[:, 1]
    tidx = sample[:, 2]
    score = _transe_sc(entity_embedding, relation_embedding, hidx, ridx, tidx)
    return score.reshape(BATCH, 1)


# speedup vs baseline: 1.3486x; 1.0066x over previous
import functools

import jax
import jax.numpy as jnp
from jax import lax
from jax.experimental import pallas as pl
from jax.experimental.pallas import tpu as pltpu
from jax.experimental.pallas import tpu_sc as plsc

GAMMA = 12.0
BATCH = 4096
DIM = 128
LANES = 16
NUM_CORES = 2
NUM_SUBCORES = 16
NW = NUM_CORES * NUM_SUBCORES
BPW = BATCH // NW
CHUNKS = DIM // LANES
STRIDE = LANES + 1
NQ = 4
QROWS = BPW // NQ


def _transe_body(entity_hbm, relation_hbm, sample_hbm,
                 out_hbm,
                 sflat, idx_q, rows_q, partials, out_v, sem_s, sem_q):
    wid = lax.axis_index("s") * NUM_CORES + lax.axis_index("c")
    base = wid * BPW

    pltpu.async_copy(sample_hbm.at[pl.ds(base * 3, BPW * 3)], sflat,
                     sem_s).wait()

    lane3 = lax.iota(jnp.int32, LANES) * 3
    for q in range(NQ):
        for col in range(3):
            for g in range(QROWS // LANES):
                flat = lane3 + ((q * QROWS + g * LANES) * 3 + col)
                idx_q[col][q][pl.ds(g * LANES, LANES)] = plsc.load_gather(
                    sflat, [flat])

    def fire(q):
        return [
            pltpu.async_copy(entity_hbm.at[idx_q[0][q]], rows_q[0][q],
                             sem_q[q]),
            pltpu.async_copy(relation_hbm.at[idx_q[1][q]], rows_q[1][q],
                             sem_q[q]),
            pltpu.async_copy(entity_hbm.at[idx_q[2][q]], rows_q[2][q],
                             sem_q[q]),
        ]

    inflight = {0: fire(0), 1: fire(1)}

    lane = lax.iota(jnp.int32, LANES)
    for q in range(NQ):
        for h in inflight.pop(q):
            h.wait()
        if q + 2 < NQ:
            inflight[q + 2] = fire(q + 2)

        h_rows, r_rows, t_rows = rows_q[0][q], rows_q[1][q], rows_q[2][q]

        def row_body(i, carry):
            acc0 = jnp.zeros((LANES,), jnp.float32)
            acc1 = jnp.zeros((LANES,), jnp.float32)
            for c in range(0, CHUNKS, 2):
                hh = h_rows[i, pl.ds(c * LANES, LANES)]
                rr = r_rows[i, pl.ds(c * LANES, LANES)]
                tt = t_rows[i, pl.ds(c * LANES, LANES)]
                acc0 = acc0 + jnp.abs(hh + rr - tt)
                hh = h_rows[i, pl.ds((c + 1) * LANES, LANES)]
                rr = r_rows[i, pl.ds((c + 1) * LANES, LANES)]
                tt = t_rows[i, pl.ds((c + 1) * LANES, LANES)]
                acc1 = acc1 + jnp.abs(hh + rr - tt)
            partials[pl.ds((q * QROWS + i) * STRIDE, LANES)] = acc0 + acc1
            return carry

        lax.fori_loop(0, QROWS, row_body, 0, unroll=2)

        for g in range(QROWS // LANES):
            rows = (lane + (q * QROWS + g * LANES)) * STRIDE
            tot = jnp.zeros((LANES,), jnp.float32)
            for c in range(LANES):
                tot = tot + plsc.load_gather(partials, [rows + c])
            out_v[pl.ds((q * QROWS + g * LANES), LANES)] = GAMMA - tot

    pltpu.sync_copy(out_v, out_hbm.at[pl.ds(base, BPW)])


_transe_sc = functools.partial(
    pl.kernel,
    mesh=plsc.VectorSubcoreMesh(core_axis_name="c", subcore_axis_name="s"),
    out_type=jax.ShapeDtypeStruct((BATCH,), jnp.float32),
    compiler_params=pltpu.CompilerParams(needs_layout_passes=False),
    scratch_types=[
        pltpu.VMEM((BPW * 3,), jnp.int32),
        [[pltpu.VMEM((QROWS,), jnp.int32) for _ in range(NQ)]
         for _ in range(3)],
        [[pltpu.VMEM((QROWS, DIM), jnp.float32) for _ in range(NQ)]
         for _ in range(3)],
        pltpu.VMEM((BPW * STRIDE,), jnp.float32),
        pltpu.VMEM((BPW,), jnp.float32),
        pltpu.SemaphoreType.DMA,
        [pltpu.SemaphoreType.DMA for _ in range(NQ)],
    ],
)(_transe_body)


@jax.jit
def kernel(sample, entity_embedding, relation_embedding):
    score = _transe_sc(entity_embedding, relation_embedding,
                       sample.reshape(BATCH * 3))
    return score.reshape(BATCH, 1)
